# baseline (speedup 1.0000x reference)
import jax
import jax.numpy as jnp
from jax import lax
from jax.experimental import pallas as pl
from jax.experimental.pallas import tpu as pltpu
from jax.experimental.pallas import tpu_sc as plsc

N = 10000
D = 128
E = 320000
P = 100000

NC, NS, L = 2, 16, 16
NW = NC * NS
NSLC = D // L
HALF = N // 2
EG = E // NC

EPT = E // NW
GCH = 400
GPIECES = ((0, 104), (104, 104), (208, 96), (304, 96))
NGCH = EPT // GCH

CH = 2000
CV = CH // L
NCH = EG // CH

PP = 102400
PB = PP // NW
GB = 128
BLK = 2000
BLKE = 2000
BLKP = 2048

_MESH = dict(core_axis_name="c", subcore_axis_name="s",
             num_cores=NC, num_subcores=NS)


def _gather_body(tab_hbm, src_hbm, out_hbm, idxb, rows, sem):
    c = lax.axis_index("c")
    s = lax.axis_index("s")
    base = (s * NC + c) * EPT

    def chunk(t, carry):
        ebase = base + t * GCH
        pltpu.sync_copy(src_hbm.at[pl.ds(ebase, GCH)], idxb)
        for o, n in GPIECES:
            pltpu.async_copy(tab_hbm.at[idxb.at[pl.ds(o, n)]],
                             rows.at[pl.ds(o, n)], sem)
        for o, n in GPIECES:
            pltpu.make_async_copy(tab_hbm.at[idxb.at[pl.ds(o, n)]],
                                  rows.at[pl.ds(o, n)], sem).wait()
        pltpu.sync_copy(rows, out_hbm.at[pl.ds(ebase, GCH)])
        return carry

    lax.fori_loop(0, NGCH, chunk, 0)


def _gather(tab, src):
    return pl.kernel(
        _gather_body,
        out_type=jax.ShapeDtypeStruct((E, D), jnp.float32),
        mesh=plsc.VectorSubcoreMesh(**_MESH),
        scratch_types=[
            pltpu.VMEM((GCH,), jnp.int32),
            pltpu.VMEM((GCH, D), jnp.float32),
            pltpu.SemaphoreType.DMA,
        ],
    )(tab, src)


def _segmax_body(pool_hbm, dst_hbm, out_hbm, acc, dstb, rows, sem):
    c = lax.axis_index("c")
    s = lax.axis_index("s")
    h = s // NSLC
    f = s % NSLC
    lo = h * HALF

    def initacc(i, carry):
        acc[i] = jnp.zeros((L,), jnp.float32)
        return carry
    lax.fori_loop(0, HALF + 1, initacc, 0, unroll=8)

    def chunk(t, carry):
        pltpu.sync_copy(dst_hbm.at[c, t], dstb)
        pltpu.async_copy(pool_hbm.at[c, t, pl.ds(0, CH), f], rows, sem).wait()

        def apply(i, carry2):
            dl = dstb[i] - lo
            okm = (dl >= 0) & (dl < HALF)
            dsel = jnp.where(okm, dl, jnp.int32(HALF))
            for lane in range(L):
                d = dsel[lane]
                r = rows[i * L + lane]
                acc[d] = jnp.maximum(acc[d], r)
            return carry2
        lax.fori_loop(0, CV, apply, 0)
        return carry

    lax.fori_loop(0, NCH, chunk, 0)
    pltpu.sync_copy(acc.at[pl.ds(0, HALF)], out_hbm.at[c, h, f])


def _segmax(pooled, dst4):
    pool5 = pooled.reshape(NC, NCH, CH, NSLC, L)
    return pl.kernel(
        _segmax_body,
        out_type=jax.ShapeDtypeStruct((NC, 2, NSLC, HALF, L), jnp.float32),
        mesh=plsc.VectorSubcoreMesh(**_MESH),
        compiler_params=pltpu.CompilerParams(use_tc_tiling_on_sc=False),
        scratch_types=[
            pltpu.VMEM((HALF + 1, L), jnp.float32),
            pltpu.VMEM((CV, L), jnp.int32),
            pltpu.VMEM((CH, L), jnp.float32),
            pltpu.SemaphoreType.DMA,
        ],
    )(pool5, dst4)


def _pred_body(g_hbm, p0_hbm, p1_hbm, out_hbm, i0, i1, rows, sem):
    c = lax.axis_index("c")
    s = lax.axis_index("s")
    base = (s * NC + c) * PB
    pltpu.sync_copy(p0_hbm.at[pl.ds(base, PB)], i0)
    pltpu.sync_copy(p1_hbm.at[pl.ds(base, PB)], i1)

    def chunk(t, carry):
        cb = t * GCH
        for o, n in GPIECES:
            pltpu.async_copy(g_hbm.at[i0.at[pl.ds(cb + o, n)]],
                             rows.at[pl.ds(o, n)], sem)
        for o, n in GPIECES:
            pltpu.make_async_copy(g_hbm.at[i0.at[pl.ds(cb + o, n)]],
                                  rows.at[pl.ds(o, n)], sem).wait()
        for o, n in GPIECES:
            pltpu.async_copy(g_hbm.at[i1.at[pl.ds(cb + o, n)]],
                             rows.at[pl.ds(o, n)], sem, add=True)
        for o, n in GPIECES:
            pltpu.make_async_copy(g_hbm.at[i1.at[pl.ds(cb + o, n)]],
                                  rows.at[pl.ds(o, n)], sem).wait()
        pltpu.sync_copy(rows, out_hbm.at[pl.ds(base + cb, GCH)])
        return carry

    lax.fori_loop(0, PB // GCH, chunk, 0)


def _pred(g, p0, p1):
    return pl.kernel(
        _pred_body,
        out_type=jax.ShapeDtypeStruct((PP, D), jnp.float32),
        mesh=plsc.VectorSubcoreMesh(**_MESH),
        scratch_types=[
            pltpu.VMEM((PB,), jnp.int32),
            pltpu.VMEM((PB,), jnp.int32),
            pltpu.VMEM((GCH, D), jnp.float32),
            pltpu.SemaphoreType.DMA,
        ],
    )(g, p0, p1)


def _head_kernel(ee_ref, w_ref, b_ref, o_ref):
    z = jnp.dot(ee_ref[...], w_ref[...],
                preferred_element_type=jnp.float32) + b_ref[...]
    col = lax.broadcasted_iota(jnp.int32, z.shape, 1)
    o_ref[...] = jnp.where(col == 0, jnp.maximum(z, 0.0), z)


def _head(ee, w16, b16):
    return pl.pallas_call(
        _head_kernel,
        grid=(PP // BLKP,),
        in_specs=[pl.BlockSpec((BLKP, D), lambda i: (i, 0)),
                  pl.BlockSpec((D, L), lambda i: (0, 0)),
                  pl.BlockSpec((1, L), lambda i: (0, 0))],
        out_specs=pl.BlockSpec((BLKP, L), lambda i: (i, 0)),
        out_shape=jax.ShapeDtypeStruct((PP, L), jnp.float32),
    )(ee, w16, b16.reshape(1, L))


def _pool_kernel(xg_ref, s_ref, w_ref, b_ref, o_ref):
    ef = xg_ref[...] * s_ref[...]
    z = jnp.dot(ef, w_ref[...], preferred_element_type=jnp.float32)
    o_ref[...] = jnp.maximum(z + b_ref[...], 0.0)


def _pool(xg, s, w, b):
    return pl.pallas_call(
        _pool_kernel,
        grid=(E // BLKE,),
        in_specs=[pl.BlockSpec((BLKE, D), lambda i: (i, 0)),
                  pl.BlockSpec((BLKE, 1), lambda i: (i, 0)),
                  pl.BlockSpec((D, D), lambda i: (0, 0)),
                  pl.BlockSpec((1, D), lambda i: (0, 0))],
        out_specs=pl.BlockSpec((BLKE, D), lambda i: (i, 0)),
        out_shape=jax.ShapeDtypeStruct((E, D), jnp.float32),
    )(xg, s.reshape(E, 1), w, b.reshape(1, D))


def _layer_kernel(x_ref, ma_ref, mb_ref, w_ref, lb_ref, h_ref):
    agg = jnp.maximum(ma_ref[...], mb_ref[...])
    hh = jnp.concatenate([x_ref[...], agg], axis=-1)
    hv = jnp.dot(hh, w_ref[...], preferred_element_type=jnp.float32)
    h_ref[...] = jnp.maximum(hv + lb_ref[...], 0.0)


def _layer(x, ma, mb, lin_w, lin_b):
    n = x.shape[0]
    return pl.pallas_call(
        _layer_kernel,
        grid=(n // BLK,),
        in_specs=[pl.BlockSpec((BLK, D), lambda i: (i, 0)),
                  pl.BlockSpec((BLK, D), lambda i: (i, 0)),
                  pl.BlockSpec((BLK, D), lambda i: (i, 0)),
                  pl.BlockSpec((2 * D, D), lambda i: (0, 0)),
                  pl.BlockSpec((1, D), lambda i: (0, 0))],
        out_specs=pl.BlockSpec((BLK, D), lambda i: (i, 0)),
        out_shape=jax.ShapeDtypeStruct((n, D), jnp.float32),
    )(x, ma, mb, lin_w, lin_b.reshape(1, D))


def _split_m(m_out):
    ma = m_out[0].transpose(0, 2, 1, 3).reshape(N, D)
    mb = m_out[1].transpose(0, 2, 1, 3).reshape(N, D)
    return ma, mb


def _conv(x, src, dst4, s, pool_w, pool_b, lin_w, lin_b):
    xg = _gather(x, src)
    pooled = _pool(xg, s, pool_w, pool_b)
    ma, mb = _split_m(_segmax(pooled, dst4))
    return _layer(x, ma, mb, lin_w, lin_b)


def kernel(x, prediction_edges, message_edges, message_edgewt,
           pool1_w, pool1_b, coef1, lin1_w, lin1_b,
           pool2_w, pool2_b, coef2, lin2_w, lin2_b,
           ewp_w, ewp_b, ep_w, ep_b):
    src = message_edges[0]
    dst4 = message_edges[1].reshape(NC, NCH, CV, L)
    s1 = 1.0 + coef1 * message_edgewt
    s2 = 1.0 + coef2 * message_edgewt

    h1 = _conv(x, src, dst4, s1, pool1_w, pool1_b, lin1_w, lin1_b)
    h2 = _conv(h1, src, dst4, s2, pool2_w, pool2_b, lin2_w, lin2_b)

    p0 = jnp.pad(prediction_edges[0], (0, PP - P))
    p1 = jnp.pad(prediction_edges[1], (0, PP - P))
    ee = _pred(h2, p0, p1)

    w16 = jnp.pad(jnp.concatenate([ewp_w, ep_w], axis=1),
                  ((0, 0), (0, L - 2)))
    b16 = jnp.pad(jnp.concatenate([ewp_b, ep_b]), (0, L - 2))
    out = _head(ee, w16, b16)
    edge_weights = out[:P, 0:1]
    edge_predictor = out[:P, 1:2]
    return (edge_weights, edge_predictor)


# speedup vs baseline: 1.1631x; 1.1631x over previous
import jax
import jax.numpy as jnp
from jax import lax
from jax.experimental import pallas as pl
from jax.experimental.pallas import tpu as pltpu
from jax.experimental.pallas import tpu_sc as plsc

N = 10000
D = 128
E = 320000
P = 100000

NC, NS, L = 2, 16, 16
NW = NC * NS
NSLC = D // L
HALF = N // 2
EG = E // NC

EPT = E // NW
GCH = 400
GPIECES = ((0, 104), (104, 104), (208, 96), (304, 96))
NGCH = EPT // GCH

CH = 800
CV = CH // L
NCH = EG // CH

PP = 102400
PB = PP // NW
GB = 128
BLK = 2000
BLKE = 2000
BLKP = 2048

_MESH = dict(core_axis_name="c", subcore_axis_name="s",
             num_cores=NC, num_subcores=NS)


def _gather_body(tab_hbm, src_hbm, out_hbm, idxb, rows, sem):
    c = lax.axis_index("c")
    s = lax.axis_index("s")
    base = (s * NC + c) * EPT

    def chunk(t, carry):
        ebase = base + t * GCH
        pltpu.sync_copy(src_hbm.at[pl.ds(ebase, GCH)], idxb)
        for o, n in GPIECES:
            pltpu.async_copy(tab_hbm.at[idxb.at[pl.ds(o, n)]],
                             rows.at[pl.ds(o, n)], sem)
        for o, n in GPIECES:
            pltpu.make_async_copy(tab_hbm.at[idxb.at[pl.ds(o, n)]],
                                  rows.at[pl.ds(o, n)], sem).wait()
        pltpu.sync_copy(rows, out_hbm.at[pl.ds(ebase, GCH)])
        return carry

    lax.fori_loop(0, NGCH, chunk, 0)


def _gather(tab, src):
    return pl.kernel(
        _gather_body,
        out_type=jax.ShapeDtypeStruct((E, D), jnp.float32),
        mesh=plsc.VectorSubcoreMesh(**_MESH),
        scratch_types=[
            pltpu.VMEM((GCH,), jnp.int32),
            pltpu.VMEM((GCH, D), jnp.float32),
            pltpu.SemaphoreType.DMA,
        ],
    )(tab, src)


def _segmax_body(pool_hbm, dst_hbm, out_hbm, acc, dstb, rows, sem,
                 dstb2, rows2, sem2):
    c = lax.axis_index("c")
    s = lax.axis_index("s")
    h = s // NSLC
    f = s % NSLC
    lo = h * HALF

    def initacc(i, carry):
        acc[i] = jnp.zeros((L,), jnp.float32)
        return carry
    lax.fori_loop(0, HALF + 1, initacc, 0, unroll=8)

    def fire(t, dstb_, rows_, sem_):
        pltpu.async_copy(dst_hbm.at[c, t], dstb_, sem_)
        pltpu.async_copy(pool_hbm.at[c, t, pl.ds(0, CH), f], rows_, sem_)

    def wait(t, dstb_, rows_, sem_):
        pltpu.make_async_copy(dst_hbm.at[c, t], dstb_, sem_).wait()
        pltpu.make_async_copy(pool_hbm.at[c, t, pl.ds(0, CH), f],
                              rows_, sem_).wait()

    def apply(dstb_, rows_):
        def body(i, carry2):
            dl = dstb_[i] - lo
            okm = (dl >= 0) & (dl < HALF)
            dsel = jnp.where(okm, dl, jnp.int32(HALF))
            for lane in range(L):
                d = dsel[lane]
                r = rows_[i * L + lane]
                acc[d] = jnp.maximum(acc[d], r)
            return carry2
        lax.fori_loop(0, CV, body, 0)

    fire(0, dstb, rows, sem)

    def chunk2(u, carry):
        ta = 2 * u
        fire(ta + 1, dstb2, rows2, sem2)
        wait(ta, dstb, rows, sem)
        apply(dstb, rows)

        @pl.when(ta + 2 < NCH)
        def _():
            fire(ta + 2, dstb, rows, sem)
        wait(ta + 1, dstb2, rows2, sem2)
        apply(dstb2, rows2)
        return carry

    lax.fori_loop(0, NCH // 2, chunk2, 0)
    pltpu.sync_copy(acc.at[pl.ds(0, HALF)], out_hbm.at[c, h, f])


def _segmax(pooled, dst4):
    pool5 = pooled.reshape(NC, NCH, CH, NSLC, L)
    return pl.kernel(
        _segmax_body,
        out_type=jax.ShapeDtypeStruct((NC, 2, NSLC, HALF, L), jnp.float32),
        mesh=plsc.VectorSubcoreMesh(**_MESH),
        compiler_params=pltpu.CompilerParams(use_tc_tiling_on_sc=False),
        scratch_types=[
            pltpu.VMEM((HALF + 1, L), jnp.float32),
            pltpu.VMEM((CV, L), jnp.int32),
            pltpu.VMEM((CH, L), jnp.float32),
            pltpu.SemaphoreType.DMA,
            pltpu.VMEM((CV, L), jnp.int32),
            pltpu.VMEM((CH, L), jnp.float32),
            pltpu.SemaphoreType.DMA,
        ],
    )(pool5, dst4)


def _pred_body(g_hbm, p0_hbm, p1_hbm, out_hbm, i0, i1, rows, sem):
    c = lax.axis_index("c")
    s = lax.axis_index("s")
    base = (s * NC + c) * PB
    pltpu.sync_copy(p0_hbm.at[pl.ds(base, PB)], i0)
    pltpu.sync_copy(p1_hbm.at[pl.ds(base, PB)], i1)

    def chunk(t, carry):
        cb = t * GCH
        for o, n in GPIECES:
            pltpu.async_copy(g_hbm.at[i0.at[pl.ds(cb + o, n)]],
                             rows.at[pl.ds(o, n)], sem)
        for o, n in GPIECES:
            pltpu.make_async_copy(g_hbm.at[i0.at[pl.ds(cb + o, n)]],
                                  rows.at[pl.ds(o, n)], sem).wait()
        for o, n in GPIECES:
            pltpu.async_copy(g_hbm.at[i1.at[pl.ds(cb + o, n)]],
                             rows.at[pl.ds(o, n)], sem, add=True)
        for o, n in GPIECES:
            pltpu.make_async_copy(g_hbm.at[i1.at[pl.ds(cb + o, n)]],
                                  rows.at[pl.ds(o, n)], sem).wait()
        pltpu.sync_copy(rows, out_hbm.at[pl.ds(base + cb, GCH)])
        return carry

    lax.fori_loop(0, PB // GCH, chunk, 0)


def _pred(g, p0, p1):
    return pl.kernel(
        _pred_body,
        out_type=jax.ShapeDtypeStruct((PP, D), jnp.float32),
        mesh=plsc.VectorSubcoreMesh(**_MESH),
        scratch_types=[
            pltpu.VMEM((PB,), jnp.int32),
            pltpu.VMEM((PB,), jnp.int32),
            pltpu.VMEM((GCH, D), jnp.float32),
            pltpu.SemaphoreType.DMA,
        ],
    )(g, p0, p1)


def _head_kernel(ee_ref, w_ref, b_ref, o_ref):
    z = jnp.dot(ee_ref[...], w_ref[...],
                preferred_element_type=jnp.float32) + b_ref[...]
    col = lax.broadcasted_iota(jnp.int32, z.shape, 1)
    o_ref[...] = jnp.where(col == 0, jnp.maximum(z, 0.0), z)


def _head(ee, w16, b16):
    return pl.pallas_call(
        _head_kernel,
        grid=(PP // BLKP,),
        in_specs=[pl.BlockSpec((BLKP, D), lambda i: (i, 0)),
                  pl.BlockSpec((D, L), lambda i: (0, 0)),
                  pl.BlockSpec((1, L), lambda i: (0, 0))],
        out_specs=pl.BlockSpec((BLKP, L), lambda i: (i, 0)),
        out_shape=jax.ShapeDtypeStruct((PP, L), jnp.float32),
    )(ee, w16, b16.reshape(1, L))


def _pool_kernel(xg_ref, s_ref, w_ref, b_ref, o_ref):
    ef = xg_ref[...] * s_ref[...]
    z = jnp.dot(ef, w_ref[...], preferred_element_type=jnp.float32)
    o_ref[...] = jnp.maximum(z + b_ref[...], 0.0)


def _pool(xg, s, w, b):
    return pl.pallas_call(
        _pool_kernel,
        grid=(E // BLKE,),
        in_specs=[pl.BlockSpec((BLKE, D), lambda i: (i, 0)),
                  pl.BlockSpec((BLKE, 1), lambda i: (i, 0)),
                  pl.BlockSpec((D, D), lambda i: (0, 0)),
                  pl.BlockSpec((1, D), lambda i: (0, 0))],
        out_specs=pl.BlockSpec((BLKE, D), lambda i: (i, 0)),
        out_shape=jax.ShapeDtypeStruct((E, D), jnp.float32),
    )(xg, s.reshape(E, 1), w, b.reshape(1, D))


def _layer_kernel(x_ref, ma_ref, mb_ref, w_ref, lb_ref, h_ref):
    agg = jnp.maximum(ma_ref[...], mb_ref[...])
    hh = jnp.concatenate([x_ref[...], agg], axis=-1)
    hv = jnp.dot(hh, w_ref[...], preferred_element_type=jnp.float32)
    h_ref[...] = jnp.maximum(hv + lb_ref[...], 0.0)


def _layer(x, ma, mb, lin_w, lin_b):
    n = x.shape[0]
    return pl.pallas_call(
        _layer_kernel,
        grid=(n // BLK,),
        in_specs=[pl.BlockSpec((BLK, D), lambda i: (i, 0)),
                  pl.BlockSpec((BLK, D), lambda i: (i, 0)),
                  pl.BlockSpec((BLK, D), lambda i: (i, 0)),
                  pl.BlockSpec((2 * D, D), lambda i: (0, 0)),
                  pl.BlockSpec((1, D), lambda i: (0, 0))],
        out_specs=pl.BlockSpec((BLK, D), lambda i: (i, 0)),
        out_shape=jax.ShapeDtypeStruct((n, D), jnp.float32),
    )(x, ma, mb, lin_w, lin_b.reshape(1, D))


def _split_m(m_out):
    ma = m_out[0].transpose(0, 2, 1, 3).reshape(N, D)
    mb = m_out[1].transpose(0, 2, 1, 3).reshape(N, D)
    return ma, mb


def _conv(x, src, dst4, s, pool_w, pool_b, lin_w, lin_b):
    xg = _gather(x, src)
    pooled = _pool(xg, s, pool_w, pool_b)
    ma, mb = _split_m(_segmax(pooled, dst4))
    return _layer(x, ma, mb, lin_w, lin_b)


def kernel(x, prediction_edges, message_edges, message_edgewt,
           pool1_w, pool1_b, coef1, lin1_w, lin1_b,
           pool2_w, pool2_b, coef2, lin2_w, lin2_b,
           ewp_w, ewp_b, ep_w, ep_b):
    src = message_edges[0]
    dst4 = message_edges[1].reshape(NC, NCH, CV, L)
    s1 = 1.0 + coef1 * message_edgewt
    s2 = 1.0 + coef2 * message_edgewt

    h1 = _conv(x, src, dst4, s1, pool1_w, pool1_b, lin1_w, lin1_b)
    h2 = _conv(h1, src, dst4, s2, pool2_w, pool2_b, lin2_w, lin2_b)

    p0 = jnp.pad(prediction_edges[0], (0, PP - P))
    p1 = jnp.pad(prediction_edges[1], (0, PP - P))
    ee = _pred(h2, p0, p1)

    w16 = jnp.pad(jnp.concatenate([ewp_w, ep_w], axis=1),
                  ((0, 0), (0, L - 2)))
    b16 = jnp.pad(jnp.concatenate([ewp_b, ep_b]), (0, L - 2))
    out = _head(ee, w16, b16)
    edge_weights = out[:P, 0:1]
    edge_predictor = out[:P, 1:2]
    return (edge_weights, edge_predictor)


# speedup vs baseline: 1.1674x; 1.0037x over previous
import jax
import jax.numpy as jnp
from jax import lax
from jax.experimental import pallas as pl
from jax.experimental.pallas import tpu as pltpu
from jax.experimental.pallas import tpu_sc as plsc

N = 10000
D = 128
E = 320000
P = 100000

NC, NS, L = 2, 16, 16
NW = NC * NS
NSLC = D // L
HALF = N // 2
EG = E // NC

EPT = E // NW
GCH = 400
GPIECES = ((0, 104), (104, 104), (208, 96), (304, 96))
GCH2 = 200
G2PIECES = ((0, 104), (104, 96))
NGCH = EPT // GCH2

CH = 800
CV = CH // L
NCH = EG // CH

PP = 102400
PB = PP // NW
GB = 128
BLK = 2000
BLKE = 2000
BLKP = 2048

_MESH = dict(core_axis_name="c", subcore_axis_name="s",
             num_cores=NC, num_subcores=NS)


def _gather_body(tab_hbm, src_hbm, out_hbm, idxb, rows, sem, isem,
                 idxb2, rows2, sem2, isem2, osem):
    c = lax.axis_index("c")
    s = lax.axis_index("s")
    base = (s * NC + c) * EPT

    def firein(t, idxb_, isem_):
        pltpu.async_copy(src_hbm.at[pl.ds(base + t * GCH2, GCH2)],
                         idxb_, isem_)

    def waitin(t, idxb_, isem_):
        pltpu.make_async_copy(src_hbm.at[pl.ds(base + t * GCH2, GCH2)],
                              idxb_, isem_).wait()

    def gath(idxb_, rows_, sem_):
        for o, n in G2PIECES:
            pltpu.async_copy(tab_hbm.at[idxb_.at[pl.ds(o, n)]],
                             rows_.at[pl.ds(o, n)], sem_)
        for o, n in G2PIECES:
            pltpu.make_async_copy(tab_hbm.at[idxb_.at[pl.ds(o, n)]],
                                  rows_.at[pl.ds(o, n)], sem_).wait()

    def out(t, rows_, sem_):
        pltpu.async_copy(rows_, out_hbm.at[pl.ds(base + t * GCH2, GCH2)],
                         sem_)

    def outwait(t, rows_, sem_):
        pltpu.make_async_copy(rows_,
                              out_hbm.at[pl.ds(base + t * GCH2, GCH2)],
                              sem_).wait()

    firein(0, idxb, isem)

    def chunk2(u, carry):
        ta = 2 * u
        firein(ta + 1, idxb2, isem2)
        waitin(ta, idxb, isem)
        gath(idxb, rows, sem)
        out(ta, rows, osem)
        waitin(ta + 1, idxb2, isem2)
        gath(idxb2, rows2, sem2)
        out(ta + 1, rows2, osem)
        outwait(ta, rows, osem)

        @pl.when(ta + 2 < NGCH)
        def _():
            firein(ta + 2, idxb, isem)
        outwait(ta + 1, rows2, osem)
        return carry

    lax.fori_loop(0, NGCH // 2, chunk2, 0)


def _gather(tab, src):
    return pl.kernel(
        _gather_body,
        out_type=jax.ShapeDtypeStruct((E, D), jnp.float32),
        mesh=plsc.VectorSubcoreMesh(**_MESH),
        scratch_types=[
            pltpu.VMEM((GCH2,), jnp.int32),
            pltpu.VMEM((GCH2, D), jnp.float32),
            pltpu.SemaphoreType.DMA,
            pltpu.SemaphoreType.DMA,
            pltpu.VMEM((GCH2,), jnp.int32),
            pltpu.VMEM((GCH2, D), jnp.float32),
            pltpu.SemaphoreType.DMA,
            pltpu.SemaphoreType.DMA,
            pltpu.SemaphoreType.DMA,
        ],
    )(tab, src)


def _segmax_body(pool_hbm, dst_hbm, out_hbm, acc, dstb, rows, sem,
                 dstb2, rows2, sem2):
    c = lax.axis_index("c")
    s = lax.axis_index("s")
    h = s // NSLC
    f = s % NSLC
    lo = h * HALF

    def initacc(i, carry):
        acc[i] = jnp.zeros((L,), jnp.float32)
        return carry
    lax.fori_loop(0, HALF + 1, initacc, 0, unroll=8)

    def fire(t, dstb_, rows_, sem_):
        pltpu.async_copy(dst_hbm.at[c, t], dstb_, sem_)
        pltpu.async_copy(pool_hbm.at[c, t, pl.ds(0, CH), f], rows_, sem_)

    def wait(t, dstb_, rows_, sem_):
        pltpu.make_async_copy(dst_hbm.at[c, t], dstb_, sem_).wait()
        pltpu.make_async_copy(pool_hbm.at[c, t, pl.ds(0, CH), f],
                              rows_, sem_).wait()

    def apply(dstb_, rows_):
        def body(i, carry2):
            dl = dstb_[i] - lo
            okm = (dl >= 0) & (dl < HALF)
            dsel = jnp.where(okm, dl, jnp.int32(HALF))
            for lane in range(L):
                d = dsel[lane]
                r = rows_[i * L + lane]
                acc[d] = jnp.maximum(acc[d], r)
            return carry2
        lax.fori_loop(0, CV, body, 0)

    fire(0, dstb, rows, sem)

    def chunk2(u, carry):
        ta = 2 * u
        fire(ta + 1, dstb2, rows2, sem2)
        wait(ta, dstb, rows, sem)
        apply(dstb, rows)

        @pl.when(ta + 2 < NCH)
        def _():
            fire(ta + 2, dstb, rows, sem)
        wait(ta + 1, dstb2, rows2, sem2)
        apply(dstb2, rows2)
        return carry

    lax.fori_loop(0, NCH // 2, chunk2, 0)
    pltpu.sync_copy(acc.at[pl.ds(0, HALF)], out_hbm.at[c, h, f])


def _segmax(pooled, dst4):
    pool5 = pooled.reshape(NC, NCH, CH, NSLC, L)
    return pl.kernel(
        _segmax_body,
        out_type=jax.ShapeDtypeStruct((NC, 2, NSLC, HALF, L), jnp.float32),
        mesh=plsc.VectorSubcoreMesh(**_MESH),
        compiler_params=pltpu.CompilerParams(use_tc_tiling_on_sc=False),
        scratch_types=[
            pltpu.VMEM((HALF + 1, L), jnp.float32),
            pltpu.VMEM((CV, L), jnp.int32),
            pltpu.VMEM((CH, L), jnp.float32),
            pltpu.SemaphoreType.DMA,
            pltpu.VMEM((CV, L), jnp.int32),
            pltpu.VMEM((CH, L), jnp.float32),
            pltpu.SemaphoreType.DMA,
        ],
    )(pool5, dst4)


def _pred_body(g_hbm, p0_hbm, p1_hbm, out_hbm, i0, i1, rows, sem,
               rows2, sem2, osem):
    c = lax.axis_index("c")
    s = lax.axis_index("s")
    base = (s * NC + c) * PB
    pltpu.sync_copy(p0_hbm.at[pl.ds(base, PB)], i0)
    pltpu.sync_copy(p1_hbm.at[pl.ds(base, PB)], i1)

    def fire(t, idx, rows_, sem_, add):
        cb = t * GCH
        for o, n in GPIECES:
            pltpu.async_copy(g_hbm.at[idx.at[pl.ds(cb + o, n)]],
                             rows_.at[pl.ds(o, n)], sem_, add=add)

    def drain(t, idx, rows_, sem_):
        cb = t * GCH
        for o, n in GPIECES:
            pltpu.make_async_copy(g_hbm.at[idx.at[pl.ds(cb + o, n)]],
                                  rows_.at[pl.ds(o, n)], sem_).wait()

    def out(t, rows_, sem_):
        pltpu.async_copy(rows_, out_hbm.at[pl.ds(base + t * GCH, GCH)], sem_)

    def outwait(t, rows_, sem_):
        pltpu.make_async_copy(rows_,
                              out_hbm.at[pl.ds(base + t * GCH, GCH)],
                              sem_).wait()

    fire(0, i0, rows, sem, False)

    def chunk2(u, carry):
        ta = 2 * u
        fire(ta + 1, i0, rows2, sem2, False)
        drain(ta, i0, rows, sem)
        fire(ta, i1, rows, sem, True)
        drain(ta, i1, rows, sem)
        out(ta, rows, osem)
        drain(ta + 1, i0, rows2, sem2)
        fire(ta + 1, i1, rows2, sem2, True)
        outwait(ta, rows, osem)

        @pl.when(ta + 2 < PB // GCH)
        def _():
            fire(ta + 2, i0, rows, sem, False)
        drain(ta + 1, i1, rows2, sem2)
        out(ta + 1, rows2, osem)
        outwait(ta + 1, rows2, osem)
        return carry

    lax.fori_loop(0, PB // GCH // 2, chunk2, 0)


def _pred(g, p0, p1):
    return pl.kernel(
        _pred_body,
        out_type=jax.ShapeDtypeStruct((PP, D), jnp.float32),
        mesh=plsc.VectorSubcoreMesh(**_MESH),
        scratch_types=[
            pltpu.VMEM((PB,), jnp.int32),
            pltpu.VMEM((PB,), jnp.int32),
            pltpu.VMEM((GCH, D), jnp.float32),
            pltpu.SemaphoreType.DMA,
            pltpu.VMEM((GCH, D), jnp.float32),
            pltpu.SemaphoreType.DMA,
            pltpu.SemaphoreType.DMA,
        ],
    )(g, p0, p1)


def _head_kernel(ee_ref, w_ref, b_ref, o_ref):
    z = jnp.dot(ee_ref[...], w_ref[...],
                preferred_element_type=jnp.float32) + b_ref[...]
    col = lax.broadcasted_iota(jnp.int32, z.shape, 1)
    o_ref[...] = jnp.where(col == 0, jnp.maximum(z, 0.0), z)


def _head(ee, w16, b16):
    return pl.pallas_call(
        _head_kernel,
        grid=(PP // BLKP,),
        in_specs=[pl.BlockSpec((BLKP, D), lambda i: (i, 0)),
                  pl.BlockSpec((D, L), lambda i: (0, 0)),
                  pl.BlockSpec((1, L), lambda i: (0, 0))],
        out_specs=pl.BlockSpec((BLKP, L), lambda i: (i, 0)),
        out_shape=jax.ShapeDtypeStruct((PP, L), jnp.float32),
    )(ee, w16, b16.reshape(1, L))


def _pool_kernel(xg_ref, s_ref, w_ref, b_ref, o_ref):
    ef = xg_ref[...] * s_ref[...]
    z = jnp.dot(ef, w_ref[...], preferred_element_type=jnp.float32)
    o_ref[...] = jnp.maximum(z + b_ref[...], 0.0)


def _pool(xg, s, w, b):
    return pl.pallas_call(
        _pool_kernel,
        grid=(E // BLKE,),
        in_specs=[pl.BlockSpec((BLKE, D), lambda i: (i, 0)),
                  pl.BlockSpec((BLKE, 1), lambda i: (i, 0)),
                  pl.BlockSpec((D, D), lambda i: (0, 0)),
                  pl.BlockSpec((1, D), lambda i: (0, 0))],
        out_specs=pl.BlockSpec((BLKE, D), lambda i: (i, 0)),
        out_shape=jax.ShapeDtypeStruct((E, D), jnp.float32),
    )(xg, s.reshape(E, 1), w, b.reshape(1, D))


def _layer_kernel(x_ref, ma_ref, mb_ref, w_ref, lb_ref, h_ref):
    agg = jnp.maximum(ma_ref[...], mb_ref[...])
    hh = jnp.concatenate([x_ref[...], agg], axis=-1)
    hv = jnp.dot(hh, w_ref[...], preferred_element_type=jnp.float32)
    h_ref[...] = jnp.maximum(hv + lb_ref[...], 0.0)


def _layer(x, ma, mb, lin_w, lin_b):
    n = x.shape[0]
    return pl.pallas_call(
        _layer_kernel,
        grid=(n // BLK,),
        in_specs=[pl.BlockSpec((BLK, D), lambda i: (i, 0)),
                  pl.BlockSpec((BLK, D), lambda i: (i, 0)),
                  pl.BlockSpec((BLK, D), lambda i: (i, 0)),
                  pl.BlockSpec((2 * D, D), lambda i: (0, 0)),
                  pl.BlockSpec((1, D), lambda i: (0, 0))],
        out_specs=pl.BlockSpec((BLK, D), lambda i: (i, 0)),
        out_shape=jax.ShapeDtypeStruct((n, D), jnp.float32),
    )(x, ma, mb, lin_w, lin_b.reshape(1, D))


def _split_m(m_out):
    ma = m_out[0].transpose(0, 2, 1, 3).reshape(N, D)
    mb = m_out[1].transpose(0, 2, 1, 3).reshape(N, D)
    return ma, mb


def _conv(x, src, dst4, s, pool_w, pool_b, lin_w, lin_b):
    xg = _gather(x, src)
    pooled = _pool(xg, s, pool_w, pool_b)
    ma, mb = _split_m(_segmax(pooled, dst4))
    return _layer(x, ma, mb, lin_w, lin_b)


def kernel(x, prediction_edges, message_edges, message_edgewt,
           pool1_w, pool1_b, coef1, lin1_w, lin1_b,
           pool2_w, pool2_b, coef2, lin2_w, lin2_b,
           ewp_w, ewp_b, ep_w, ep_b):
    src = message_edges[0]
    dst4 = message_edges[1].reshape(NC, NCH, CV, L)
    s1 = 1.0 + coef1 * message_edgewt
    s2 = 1.0 + coef2 * message_edgewt

    h1 = _conv(x, src, dst4, s1, pool1_w, pool1_b, lin1_w, lin1_b)
    h2 = _conv(h1, src, dst4, s2, pool2_w, pool2_b, lin2_w, lin2_b)

    p0 = jnp.pad(prediction_edges[0], (0, PP - P))
    p1 = jnp.pad(prediction_edges[1], (0, PP - P))
    ee = _pred(h2, p0, p1)

    w16 = jnp.pad(jnp.concatenate([ewp_w, ep_w], axis=1),
                  ((0, 0), (0, L - 2)))
    b16 = jnp.pad(jnp.concatenate([ewp_b, ep_b]), (0, L - 2))
    out = _head(ee, w16, b16)
    edge_weights = out[:P, 0:1]
    edge_predictor = out[:P, 1:2]
    return (edge_weights, edge_predictor)
